# Initial kernel scaffold; baseline (speedup 1.0000x reference)
import jax
import jax.numpy as jnp
from jax.experimental import pallas as pl


def kernel(x, edge_index, W_head, W_out):
    raise NotImplementedError("write your pallas kernel here")


# speedup vs baseline: 9.0721x; 9.0721x over previous
import functools

import jax
import jax.numpy as jnp
from jax import lax
from jax.experimental import pallas as pl
from jax.experimental.pallas import tpu as pltpu
from jax.experimental.pallas import tpu_sc as plsc

N = 10000
E = 320000
D = 128

NC = 2
NS = 16
NW = NC * NS
EPW = E // NW
CHUNK = 80
NCHUNK = EPW // CHUNK
NPAD = 10240
RPT = NPAD // NS


def _prep_body(x_ref, wh_ref, w_ref):
    x = x_ref[...]
    a = lax.dot_general(x, wh_ref[...], (((1,), (1,)), ((), ())),
                        preferred_element_type=jnp.float32)
    scores = jnp.sum(a * x, axis=1, keepdims=True) / jnp.sqrt(jnp.float32(D))
    m = jnp.max(scores)
    e = jnp.exp(scores - m)
    attn = e / jnp.sum(e)
    w_ref[...] = x * attn


_prep = pl.pallas_call(
    _prep_body,
    out_shape=jax.ShapeDtypeStruct((N, D), jnp.float32),
)


_sc_mesh = plsc.VectorSubcoreMesh(core_axis_name="c", subcore_axis_name="s")


@functools.partial(
    pl.kernel,
    out_type=jax.ShapeDtypeStruct((NC, NPAD, D), jnp.float32),
    mesh=_sc_mesh,
    scratch_types=[
        pltpu.VMEM((CHUNK,), jnp.int32),
        pltpu.VMEM((CHUNK,), jnp.int32),
        pltpu.VMEM((CHUNK, D), jnp.float32),
        pltpu.VMEM_SHARED((NPAD, D), jnp.float32),
        pltpu.SemaphoreType.DMA,
    ],
)
def _sc_scatter(src_hbm, dst_hbm, w_hbm, zero_hbm, out_hbm,
                src_v, dst_v, rows_v, acc_sh, sem):
    c = lax.axis_index("c")
    s = lax.axis_index("s")
    pltpu.sync_copy(zero_hbm.at[pl.ds(s * RPT, RPT)],
                    acc_sh.at[pl.ds(s * RPT, RPT)])
    plsc.subcore_barrier()

    base0 = (s * NC + c) * EPW

    def body(i, _):
        base = base0 + i * CHUNK
        pltpu.sync_copy(src_hbm.at[pl.ds(base, CHUNK)], src_v)
        pltpu.sync_copy(dst_hbm.at[pl.ds(base, CHUNK)], dst_v)
        pltpu.async_copy(w_hbm.at[src_v], rows_v, sem).wait()
        pltpu.sync_copy(rows_v, acc_sh.at[dst_v], add=True)
        return ()

    lax.fori_loop(0, NCHUNK, body, ())
    plsc.subcore_barrier()
    pltpu.sync_copy(acc_sh.at[pl.ds(s * RPT, RPT)],
                    out_hbm.at[c, pl.ds(s * RPT, RPT)])


def _out_body(p_ref, wo_ref, o_ref):
    h = p_ref[0, :N] + p_ref[1, :N]
    o = jnp.dot(h, wo_ref[...], preferred_element_type=jnp.float32)
    o_ref[...] = jnp.maximum(o, 0.0)


_finish = pl.pallas_call(
    _out_body,
    out_shape=jax.ShapeDtypeStruct((N, D), jnp.float32),
)


def kernel(x, edge_index, W_head, W_out):
    w = _prep(x, W_head)
    src = edge_index[0]
    dst = edge_index[1]
    zeros = jnp.zeros((NPAD, D), jnp.float32)
    parts = _sc_scatter(src, dst, w, zeros)
    return _finish(parts, W_out)


# speedup vs baseline: 19.1393x; 2.1097x over previous
import functools

import jax
import jax.numpy as jnp
from jax import lax
from jax.experimental import pallas as pl
from jax.experimental.pallas import tpu as pltpu
from jax.experimental.pallas import tpu_sc as plsc

N = 10000
E = 320000
D = 128

NC = 2
NS = 16
NW = NC * NS
EPW = E // NW
CHUNK = 80
NCHUNK = EPW // CHUNK
NPAD = 10240
RPT = NPAD // NS


def _prep_body(x_ref, wh_ref, w_ref):
    x = x_ref[...]
    a = lax.dot_general(x, wh_ref[...], (((1,), (1,)), ((), ())),
                        preferred_element_type=jnp.float32)
    scores = jnp.sum(a * x, axis=1, keepdims=True) / jnp.sqrt(jnp.float32(D))
    m = jnp.max(scores)
    e = jnp.exp(scores - m)
    attn = e / jnp.sum(e)
    w_ref[...] = x * attn


_prep = pl.pallas_call(
    _prep_body,
    out_shape=jax.ShapeDtypeStruct((N, D), jnp.float32),
)


_sc_mesh = plsc.VectorSubcoreMesh(core_axis_name="c", subcore_axis_name="s")


NBUF = 2


@functools.partial(
    pl.kernel,
    out_type=jax.ShapeDtypeStruct((NC, NPAD, D), jnp.float32),
    mesh=_sc_mesh,
    scratch_types=[
        pltpu.VMEM((NCHUNK, CHUNK), jnp.int32),
        [pltpu.VMEM((1, CHUNK), jnp.int32)] * NBUF,
        pltpu.VMEM((NBUF, CHUNK, D), jnp.float32),
        pltpu.VMEM_SHARED((NPAD, D), jnp.float32),
        [pltpu.SemaphoreType.DMA] * NBUF,
        [pltpu.SemaphoreType.DMA] * NBUF,
    ],
)
def _sc_scatter(src_hbm, dstc_hbm, w_hbm, zero_hbm, out_hbm,
                src_v, dst_bufs, rows_v, acc_sh, gsems, dsems):
    c = lax.axis_index("c")
    s = lax.axis_index("s")
    wid = s * NC + c

    pltpu.sync_copy(src_hbm.at[wid], src_v)
    gbase = wid * NCHUNK

    def start_gather(chunk, r):
        return pltpu.async_copy(w_hbm.at[src_v.at[chunk]], rows_v.at[r],
                                gsems[r])

    def start_dst(chunk, r):
        return pltpu.async_copy(dstc_hbm.at[gbase + chunk], dst_bufs[r],
                                dsems[r])

    def scat(r):
        pltpu.sync_copy(rows_v.at[r], acc_sh.at[dst_bufs[r].at[0]],
                        add=True)

    gw = [start_gather(r, r) for r in range(NBUF)]
    dw = [start_dst(r, r) for r in range(NBUF)]
    pltpu.sync_copy(zero_hbm.at[pl.ds(s * RPT, RPT)],
                    acc_sh.at[pl.ds(s * RPT, RPT)])
    plsc.subcore_barrier()

    def body(t, _):
        for r in range(NBUF):
            chunk = t * NBUF + r
            gw[r].wait()
            dw[r].wait()
            scat(r)
            start_gather(chunk + NBUF, r)
            start_dst(chunk + NBUF, r)
        return ()

    npair = (NCHUNK - NBUF - 1) // NBUF
    lax.fori_loop(0, npair, body, ())
    for r in range(NBUF):
        gw[r].wait()
        dw[r].wait()
        scat(r)
    start_gather(NCHUNK - 1, 0).wait()
    start_dst(NCHUNK - 1, 0).wait()
    scat(0)

    plsc.subcore_barrier()
    pltpu.sync_copy(acc_sh.at[pl.ds(s * RPT, RPT)],
                    out_hbm.at[c, pl.ds(s * RPT, RPT)])


def _out_body(p_ref, wo_ref, o_ref):
    h = p_ref[0, :N] + p_ref[1, :N]
    o = jnp.dot(h, wo_ref[...], preferred_element_type=jnp.float32)
    o_ref[...] = jnp.maximum(o, 0.0)


_finish = pl.pallas_call(
    _out_body,
    out_shape=jax.ShapeDtypeStruct((N, D), jnp.float32),
)


def kernel(x, edge_index, W_head, W_out):
    w = _prep(x, W_head)
    src = edge_index[0].reshape(NW, NCHUNK, CHUNK)
    dst = edge_index[1].reshape(NW * NCHUNK, 1, CHUNK)
    zeros = jnp.zeros((NPAD, D), jnp.float32)
    parts = _sc_scatter(src, dst, w, zeros)
    return _finish(parts, W_out)


# speedup vs baseline: 22.3711x; 1.1689x over previous
import functools

import jax
import jax.numpy as jnp
from jax import lax
from jax.experimental import pallas as pl
from jax.experimental.pallas import tpu as pltpu
from jax.experimental.pallas import tpu_sc as plsc

N = 10000
E = 320000
D = 128

NC = 2
NS = 16
NW = NC * NS
EPW = E // NW
CHUNK = 80
NCHUNK = EPW // CHUNK
NPAD = 10240
RPT = NPAD // NS


def _prep_body(x_ref, wh_ref, w_ref):
    x = x_ref[...]
    a = lax.dot_general(x, wh_ref[...], (((1,), (1,)), ((), ())),
                        preferred_element_type=jnp.float32)
    scores = jnp.sum(a * x, axis=1, keepdims=True) / jnp.sqrt(jnp.float32(D))
    m = jnp.max(scores)
    e = jnp.exp(scores - m)
    attn = e / jnp.sum(e)
    w_ref[...] = x * attn


_prep = pl.pallas_call(
    _prep_body,
    out_shape=jax.ShapeDtypeStruct((N, D), jnp.float32),
)


_sc_mesh = plsc.VectorSubcoreMesh(core_axis_name="c", subcore_axis_name="s")


NBUF = 3


@functools.partial(
    pl.kernel,
    out_type=jax.ShapeDtypeStruct((NC, NPAD, D), jnp.float32),
    mesh=_sc_mesh,
    scratch_types=[
        pltpu.VMEM((EPW,), jnp.int32),
        [pltpu.VMEM((1, CHUNK), jnp.int32)] * NBUF,
        pltpu.VMEM((NBUF, CHUNK, D), jnp.float32),
        pltpu.VMEM_SHARED((NPAD, D), jnp.float32),
        [pltpu.SemaphoreType.DMA] * NBUF,
        [pltpu.SemaphoreType.DMA] * NBUF,
    ],
)
def _sc_scatter(src_hbm, dstc_hbm, w_hbm, out_hbm,
                src_v, dst_bufs, rows_v, acc_sh, gsems, dsems):
    c = lax.axis_index("c")
    s = lax.axis_index("s")
    wid = s * NC + c

    pltpu.sync_copy(src_hbm.at[pl.ds(pl.multiple_of(wid * EPW, 8), EPW)],
                    src_v)
    gbase = wid * NCHUNK

    def start_gather(chunk, r):
        off = pl.multiple_of(chunk * CHUNK, 8)
        return pltpu.async_copy(w_hbm.at[src_v.at[pl.ds(off, CHUNK)]],
                                rows_v.at[r], gsems[r])

    def start_dst(chunk, r):
        return pltpu.async_copy(dstc_hbm.at[gbase + chunk], dst_bufs[r],
                                dsems[r])

    def scat(r):
        pltpu.sync_copy(rows_v.at[r], acc_sh.at[dst_bufs[r].at[0]],
                        add=True)

    zero16 = jnp.zeros((16,), jnp.float32)

    def zfill(i, _):
        for j in range(D // 16):
            rows_v[0, i, pl.ds(j * 16, 16)] = zero16
        return ()

    lax.fori_loop(0, CHUNK, zfill, ())
    for k in range(RPT // CHUNK):
        pltpu.sync_copy(rows_v.at[0],
                        acc_sh.at[pl.ds(s * RPT + k * CHUNK, CHUNK)])

    gw = [start_gather(r, r) for r in range(NBUF)]
    dw = [start_dst(r, r) for r in range(NBUF)]
    plsc.subcore_barrier()

    def body(t, _):
        for r in range(NBUF):
            chunk = t * NBUF + r
            gw[r].wait()
            dw[r].wait()
            scat(r)
            start_gather(chunk + NBUF, r)
            start_dst(chunk + NBUF, r)
        return ()

    npair = (NCHUNK - NBUF - 1) // NBUF
    lax.fori_loop(0, npair, body, ())
    for r in range(NBUF):
        gw[r].wait()
        dw[r].wait()
        scat(r)
    for tchunk in range(NBUF * (npair + 1), NCHUNK):
        start_gather(tchunk, 0).wait()
        start_dst(tchunk, 0).wait()
        scat(0)

    plsc.subcore_barrier()
    pltpu.sync_copy(acc_sh.at[pl.ds(s * RPT, RPT)],
                    out_hbm.at[c, pl.ds(s * RPT, RPT)])


def _out_body(p_ref, wo_ref, o_ref):
    h = p_ref[0, :N] + p_ref[1, :N]
    o = jnp.dot(h, wo_ref[...], preferred_element_type=jnp.float32)
    o_ref[...] = jnp.maximum(o, 0.0)


_finish = pl.pallas_call(
    _out_body,
    out_shape=jax.ShapeDtypeStruct((N, D), jnp.float32),
)


def kernel(x, edge_index, W_head, W_out):
    w = _prep(x, W_head)
    src = edge_index[0]
    dst = edge_index[1].reshape(NW * NCHUNK, 1, CHUNK)
    parts = _sc_scatter(src, dst, w)
    return _finish(parts, W_out)


# speedup vs baseline: 22.9480x; 1.0258x over previous
import functools

import jax
import jax.numpy as jnp
from jax import lax
from jax.experimental import pallas as pl
from jax.experimental.pallas import tpu as pltpu
from jax.experimental.pallas import tpu_sc as plsc

N = 10000
E = 320000
D = 128

NC = 2
NS = 16
NW = NC * NS
EPW = E // NW
CHUNK = 80
NCHUNK = EPW // CHUNK
NPAD = 10240
RPT = NPAD // NS


def _prep_body(x_ref, wh_ref, w_ref):
    x = x_ref[...]
    a = lax.dot_general(x, wh_ref[...], (((1,), (1,)), ((), ())),
                        preferred_element_type=jnp.float32)
    scores = jnp.sum(a * x, axis=1, keepdims=True) / jnp.sqrt(jnp.float32(D))
    m = jnp.max(scores)
    e = jnp.exp(scores - m)
    attn = e / jnp.sum(e)
    w_ref[...] = x * attn


_prep = pl.pallas_call(
    _prep_body,
    out_shape=jax.ShapeDtypeStruct((N, D), jnp.float32),
)


_sc_mesh = plsc.VectorSubcoreMesh(core_axis_name="c", subcore_axis_name="s")


NBUF = 3


@functools.partial(
    pl.kernel,
    out_type=jax.ShapeDtypeStruct((NC, NPAD, D), jnp.float32),
    mesh=_sc_mesh,
    scratch_types=[
        pltpu.VMEM((EPW,), jnp.int32),
        [pltpu.VMEM((CHUNK,), jnp.int32)] * NBUF,
        pltpu.VMEM((NBUF, CHUNK, D), jnp.float32),
        pltpu.VMEM_SHARED((NPAD, D), jnp.float32),
        [pltpu.SemaphoreType.DMA] * NBUF,
        [pltpu.SemaphoreType.DMA] * NBUF,
    ],
)
def _sc_scatter(src_hbm, dstc_hbm, w_hbm, out_hbm,
                src_v, dst_bufs, rows_v, acc_sh, gsems, dsems):
    c = lax.axis_index("c")
    s = lax.axis_index("s")
    wid = s * NC + c

    ebase = pl.multiple_of(wid * EPW, 8)
    pltpu.sync_copy(src_hbm.at[pl.ds(ebase, EPW)], src_v)

    def start_gather(chunk, r):
        off = pl.multiple_of(chunk * CHUNK, 8)
        return pltpu.async_copy(w_hbm.at[src_v.at[pl.ds(off, CHUNK)]],
                                rows_v.at[r], gsems[r])

    def start_dst(chunk, r):
        off = pl.multiple_of(wid * EPW + chunk * CHUNK, 8)
        return pltpu.async_copy(dstc_hbm.at[pl.ds(off, CHUNK)], dst_bufs[r],
                                dsems[r])

    def scat(r):
        pltpu.sync_copy(rows_v.at[r], acc_sh.at[dst_bufs[r]], add=True)

    zero16 = jnp.zeros((16,), jnp.float32)

    def zfill(i, _):
        for j in range(D // 16):
            rows_v[0, i, pl.ds(j * 16, 16)] = zero16
        return ()

    lax.fori_loop(0, CHUNK, zfill, ())
    for k in range(RPT // CHUNK):
        pltpu.sync_copy(rows_v.at[0],
                        acc_sh.at[pl.ds(s * RPT + k * CHUNK, CHUNK)])

    gw = [start_gather(r, r) for r in range(NBUF)]
    dw = [start_dst(r, r) for r in range(NBUF)]
    plsc.subcore_barrier()

    def body(t, _):
        for r in range(NBUF):
            chunk = t * NBUF + r
            gw[r].wait()
            dw[r].wait()
            scat(r)
            start_gather(chunk + NBUF, r)
            start_dst(chunk + NBUF, r)
        return ()

    npair = (NCHUNK - NBUF - 1) // NBUF
    lax.fori_loop(0, npair, body, ())
    for r in range(NBUF):
        gw[r].wait()
        dw[r].wait()
        scat(r)
    for tchunk in range(NBUF * (npair + 1), NCHUNK):
        start_gather(tchunk, 0).wait()
        start_dst(tchunk, 0).wait()
        scat(0)

    plsc.subcore_barrier()
    pltpu.sync_copy(acc_sh.at[pl.ds(s * RPT, RPT)],
                    out_hbm.at[c, pl.ds(s * RPT, RPT)])


def _out_body(p_ref, wo_ref, o_ref):
    h = p_ref[0, :N] + p_ref[1, :N]
    o = jnp.dot(h, wo_ref[...], preferred_element_type=jnp.float32)
    o_ref[...] = jnp.maximum(o, 0.0)


_finish = pl.pallas_call(
    _out_body,
    out_shape=jax.ShapeDtypeStruct((N, D), jnp.float32),
)


def kernel(x, edge_index, W_head, W_out):
    w = _prep(x, W_head)
    src = edge_index[0]
    dst = edge_index[1]
    parts = _sc_scatter(src, dst, w)
    return _finish(parts, W_out)
